# baseline (speedup 1.0000x reference)
import functools

import jax
import jax.numpy as jnp
from jax import lax
from jax.experimental import pallas as pl
from jax.experimental.pallas import tpu as pltpu
from jax.experimental.pallas import tpu_sc as plsc

VOCAB = 1000
EMBED_DIM = 64
HIDDEN_DIM = 32


def _table_body(emb_ref, w1_ref, b1_ref, w2_ref, b2_ref, out_ref):
    h = lax.dot_general(
        emb_ref[...], w1_ref[...], (((1,), (0,)), ((), ())),
        preferred_element_type=jnp.float32)
    h = jnp.maximum(h + b1_ref[...], 0.0)
    out_ref[...] = lax.dot_general(
        h, w2_ref[...], (((1,), (0,)), ((), ())),
        preferred_element_type=jnp.float32) + b2_ref[...]


def _compute_table(emb, W1, b1, W2, b2):
    return pl.pallas_call(
        _table_body,
        out_shape=jax.ShapeDtypeStruct((VOCAB, VOCAB), jnp.float32),
    )(emb, W1, b1.reshape(1, HIDDEN_DIM), W2, b2.reshape(1, VOCAB))


@functools.cache
def _make_gather(B, D):
    info = plsc.get_sparse_core_info()
    NC, NS = info.num_cores, info.num_subcores
    NW = NC * NS
    b_per_w = B // NW
    assert B % NW == 0 and b_per_w % 8 == 0
    C = 80
    n_chunks = b_per_w // C
    assert b_per_w % C == 0
    mesh = plsc.VectorSubcoreMesh(core_axis_name="c", subcore_axis_name="s")

    @functools.partial(
        pl.kernel, mesh=mesh,
        compiler_params=pltpu.CompilerParams(use_tc_tiling_on_sc=False),
        out_type=jax.ShapeDtypeStruct((B, D), jnp.float32),
        scratch_types=[
            pltpu.VMEM((b_per_w,), jnp.int32),
            pltpu.VMEM((C, D), jnp.float32),
            pltpu.SemaphoreType.DMA,
        ],
    )
    def gather(table_hbm, idx_hbm, out_hbm, idx_v, rows_v, sem):
        wid = lax.axis_index("s") * NC + lax.axis_index("c")
        base = wid * b_per_w
        pltpu.sync_copy(idx_hbm.at[pl.ds(base, b_per_w)], idx_v)
        for j in range(n_chunks):
            pltpu.async_copy(
                table_hbm.at[idx_v.at[pl.ds(j * C, C)]], rows_v, sem).wait()
            pltpu.sync_copy(rows_v, out_hbm.at[pl.ds(base + j * C, C)])

    return gather


def kernel(x, emb, W1, b1, W2, b2):
    table = _compute_table(emb, W1, b1, W2, b2)
    Bt, L = x.shape
    flat = x.reshape(-1).astype(jnp.int32)
    out = _make_gather(Bt * L, VOCAB)(table, flat)
    return out.reshape(Bt, L, VOCAB)


# speedup vs baseline: 1.1194x; 1.1194x over previous
import functools

import jax
import jax.numpy as jnp
from jax import lax
from jax.experimental import pallas as pl
from jax.experimental.pallas import tpu as pltpu
from jax.experimental.pallas import tpu_sc as plsc

VOCAB = 1000
EMBED_DIM = 64
HIDDEN_DIM = 32


def _table_body(emb_ref, w1_ref, b1_ref, w2_ref, b2_ref, out_ref):
    h = lax.dot_general(
        emb_ref[...], w1_ref[...], (((1,), (0,)), ((), ())),
        preferred_element_type=jnp.float32)
    h = jnp.maximum(h + b1_ref[...], 0.0)
    out_ref[...] = lax.dot_general(
        h, w2_ref[...], (((1,), (0,)), ((), ())),
        preferred_element_type=jnp.float32) + b2_ref[...]


def _compute_table(emb, W1, b1, W2, b2):
    return pl.pallas_call(
        _table_body,
        out_shape=jax.ShapeDtypeStruct((VOCAB, VOCAB), jnp.float32),
    )(emb, W1, b1.reshape(1, HIDDEN_DIM), W2, b2.reshape(1, VOCAB))


@functools.cache
def _make_gather(B, D, V):
    info = plsc.get_sparse_core_info()
    NC, NS = info.num_cores, info.num_subcores
    NW = NC * NS
    b_per_w = B // NW
    assert B % NW == 0 and b_per_w % 8 == 0
    C = 32
    n_chunks = b_per_w // C
    assert b_per_w % C == 0
    n_stagers = 8
    v_per_s = V // n_stagers
    assert V % n_stagers == 0
    mesh = plsc.VectorSubcoreMesh(core_axis_name="c", subcore_axis_name="s")

    @functools.partial(
        pl.kernel, mesh=mesh,
        compiler_params=pltpu.CompilerParams(use_tc_tiling_on_sc=False),
        out_type=jax.ShapeDtypeStruct((B, D), jnp.float32),
        scratch_types=[
            pltpu.VMEM_SHARED((V, D), jnp.float32),
            pltpu.VMEM((b_per_w,), jnp.int32),
            pltpu.VMEM((2, C, D), jnp.float32),
            pltpu.SemaphoreType.DMA,
            pltpu.SemaphoreType.DMA,
            pltpu.SemaphoreType.DMA,
            pltpu.SemaphoreType.DMA,
        ],
    )
    def gather(table_hbm, idx_hbm, out_hbm, table_sh, idx_v, rows_v,
               gsem0, gsem1, ssem0, ssem1):
        sid = lax.axis_index("s")
        wid = sid * NC + lax.axis_index("c")
        base = wid * b_per_w
        @pl.when(sid < n_stagers)
        def _():
            pltpu.sync_copy(table_hbm.at[pl.ds(sid * v_per_s, v_per_s)],
                            table_sh.at[pl.ds(sid * v_per_s, v_per_s)])
        pltpu.sync_copy(idx_hbm.at[pl.ds(base, b_per_w)], idx_v)
        plsc.subcore_barrier()

        gsems = (gsem0, gsem1)
        ssems = (ssem0, ssem1)

        def gather_start(j, b):
            return pltpu.async_copy(
                table_sh.at[idx_v.at[pl.ds(j * C, C)]], rows_v.at[b],
                gsems[b])

        def store_start(j, b):
            return pltpu.async_copy(
                rows_v.at[b], out_hbm.at[pl.ds(base + j * C, C)], ssems[b])

        gh = [gather_start(0, 0), gather_start(1, 1)]
        sh = [None, None]
        for j in range(n_chunks):
            b = j % 2
            gh[b].wait()
            sh[b] = store_start(j, b)
            if j + 2 < n_chunks:
                sh[b].wait()
                gh[b] = gather_start(j + 2, b)
        sh[(n_chunks - 2) % 2].wait()
        sh[(n_chunks - 1) % 2].wait()

    return gather


def kernel(x, emb, W1, b1, W2, b2):
    table = _compute_table(emb, W1, b1, W2, b2)
    Bt, L = x.shape
    flat = x.reshape(-1).astype(jnp.int32)
    out = _make_gather(Bt * L, VOCAB, VOCAB)(table, flat)
    return out.reshape(Bt, L, VOCAB)
